# baseline (speedup 1.0000x reference)
import jax
import jax.numpy as jnp
from jax.experimental import pallas as pl
from jax.experimental.pallas import tpu as pltpu


def _logits_tile(xt_ref, w_ref, b_ref, out_ref):
    acc = jax.lax.dot_general(
        w_ref[...], xt_ref[...],
        dimension_numbers=(((1,), (0,)), ((), ())),
        preferred_element_type=jnp.float32,
    )
    out_ref[...] = acc + b_ref[...]


@jax.jit
def _lsh_logits(inputs, W, b):
    batch, d = inputs.shape
    n = W.shape[0]
    block_n = 3072
    xt = inputs.T
    bcol = b.reshape(n, 1)
    grid = (pl.cdiv(n, block_n),)
    out_t = pl.pallas_call(
        _logits_tile,
        grid=grid,
        in_specs=[
            pl.BlockSpec((d, batch), lambda j: (0, 0)),
            pl.BlockSpec((block_n, d), lambda j: (j, 0)),
            pl.BlockSpec((block_n, 1), lambda j: (j, 0)),
        ],
        out_specs=pl.BlockSpec((block_n, batch), lambda j: (j, 0)),
        out_shape=jax.ShapeDtypeStruct((n, batch), jnp.float32),
        compiler_params=pltpu.CompilerParams(
            dimension_semantics=("arbitrary",),
        ),
    )(xt, W, bcol)
    return out_t.T


def kernel(inputs, labels, freeze, slide, W, b):
    return _lsh_logits(inputs, W, b)


# speedup vs baseline: 1.0183x; 1.0183x over previous
import jax
import jax.numpy as jnp
from jax.experimental import pallas as pl
from jax.experimental.pallas import tpu as pltpu


def _logits_tile(x_ref, w_ref, b_ref, out_ref):
    acc = jax.lax.dot_general(
        w_ref[...], x_ref[...],
        dimension_numbers=(((1,), (1,)), ((), ())),
        preferred_element_type=jnp.float32,
    )
    out_ref[...] = acc + b_ref[...]


@jax.jit
def _lsh_logits(inputs, W, b):
    batch, d = inputs.shape
    n = W.shape[0]
    block_n = 4096
    bcol = b.reshape(n, 1)
    grid = (pl.cdiv(n, block_n),)
    out_t = pl.pallas_call(
        _logits_tile,
        grid=grid,
        in_specs=[
            pl.BlockSpec((batch, d), lambda j: (0, 0)),
            pl.BlockSpec((block_n, d), lambda j: (j, 0)),
            pl.BlockSpec((block_n, 1), lambda j: (j, 0)),
        ],
        out_specs=pl.BlockSpec((block_n, batch), lambda j: (j, 0)),
        out_shape=jax.ShapeDtypeStruct((n, batch), jnp.float32),
        compiler_params=pltpu.CompilerParams(
            dimension_semantics=("arbitrary",),
            vmem_limit_bytes=67108864,
        ),
    )(inputs, W, bcol)
    return out_t.T


def kernel(inputs, labels, freeze, slide, W, b):
    return _lsh_logits(inputs, W, b)


# speedup vs baseline: 1.0313x; 1.0128x over previous
import jax
import jax.numpy as jnp
from jax.experimental import pallas as pl
from jax.experimental.pallas import tpu as pltpu

_BN = 4096
_H = _BN // 2


def _make_body(batch, d, n):
    n_steps = pl.cdiv(n, _BN)
    tail = n - (n_steps - 1) * _BN

    def body(x_ref, w_ref, b_ref, out_hbm, o_buf, o_sem):
        j = pl.program_id(0)
        slot0 = jax.lax.rem(2 * j, 4)
        slot1 = jax.lax.rem(2 * j + 1, 4)

        def copy(step, slot_idx, rows, off):
            return pltpu.make_async_copy(
                o_buf.at[slot_idx, pl.ds(0, rows), :],
                out_hbm.at[pl.ds(step * _BN + off, rows), :],
                o_sem.at[slot_idx],
            )

        x = x_ref[...]

        @pl.when(j >= 2)
        def _():
            copy(j - 2, slot0, _H, 0).wait()

        acc0 = jax.lax.dot_general(
            w_ref[pl.ds(0, _H), :], x,
            dimension_numbers=(((1,), (1,)), ((), ())),
            preferred_element_type=jnp.float32,
        )
        o_buf[slot0] = acc0 + b_ref[pl.ds(0, _H), :]

        @pl.when(j < n_steps - 1)
        def _():
            copy(j, slot0, _H, 0).start()

        @pl.when(j == n_steps - 1)
        def _():
            copy(j, slot0, tail, 0).start()

        @pl.when(j >= 2)
        def _():
            copy(j - 2, slot1, _H, _H).wait()

        acc1 = jax.lax.dot_general(
            w_ref[pl.ds(_H, _H), :], x,
            dimension_numbers=(((1,), (1,)), ((), ())),
            preferred_element_type=jnp.float32,
        )
        o_buf[slot1] = acc1 + b_ref[pl.ds(_H, _H), :]

        @pl.when(j < n_steps - 1)
        def _():
            copy(j, slot1, _H, _H).start()

        @pl.when(j == n_steps - 1)
        def _():
            copy(j - 1, jax.lax.rem(2 * (j - 1), 4), _H, 0).wait()
            copy(j - 1, jax.lax.rem(2 * (j - 1) + 1, 4), _H, _H).wait()
            copy(j, slot0, tail, 0).wait()

    return body, n_steps


@jax.jit
def _lsh_logits(inputs, W, b):
    batch, d = inputs.shape
    n = W.shape[0]
    body, n_steps = _make_body(batch, d, n)
    bcol = b.reshape(n, 1)
    out_t = pl.pallas_call(
        body,
        grid=(n_steps,),
        in_specs=[
            pl.BlockSpec((batch, d), lambda j: (0, 0)),
            pl.BlockSpec((_BN, d), lambda j: (j, 0)),
            pl.BlockSpec((_BN, 1), lambda j: (j, 0)),
        ],
        out_specs=pl.BlockSpec(memory_space=pltpu.MemorySpace.HBM),
        out_shape=jax.ShapeDtypeStruct((n, batch), jnp.float32),
        scratch_shapes=[
            pltpu.VMEM((4, _H, batch), jnp.float32),
            pltpu.SemaphoreType.DMA((4,)),
        ],
        compiler_params=pltpu.CompilerParams(
            dimension_semantics=("arbitrary",),
            vmem_limit_bytes=67108864,
        ),
    )(inputs, W, bcol)
    return out_t.T


def kernel(inputs, labels, freeze, slide, W, b):
    return _lsh_logits(inputs, W, b)
